# Initial kernel scaffold; baseline (speedup 1.0000x reference)
import jax
import jax.numpy as jnp
from jax.experimental import pallas as pl


def kernel(x, edge_index, edge_attr, W1, b1, W2, b2):
    raise NotImplementedError("write your pallas kernel here")


# speedup vs baseline: 4.5353x; 4.5353x over previous
import functools

import jax
import jax.numpy as jnp
from jax import lax
from jax.experimental import pallas as pl
from jax.experimental.pallas import tpu as pltpu
from jax.experimental.pallas import tpu_sc as plsc

N = 10000
E = 320000
F_IN = 128
LATENT = 128
CLASSES = 16

NC = 2
NS = 16
NW = NC * NS
EPW = E // NW
K = 80
NCHUNK = EPW // K

ROWS_T = 624
ROWS_LAST = N - 15 * ROWS_T


def _broadcast_lane(vec, e):
    idx = jnp.full((16, 1), e, dtype=jnp.int32)
    dn = lax.GatherDimensionNumbers(
        offset_dims=(), collapsed_slice_dims=(0,), start_index_map=(0,))
    return lax.gather(vec, idx, dn, slice_sizes=(1,),
                      mode=lax.GatherScatterMode.PROMISE_IN_BOUNDS)


def _make_sc_aggregate(feat):
    nfv = feat // 16

    def body(h_hbm, src_hbm, dst_hbm, w_hbm, zeros_hbm, out_hbm,
             acc, src_v, dst_v, w_v, rows_v, sem):
        c = lax.axis_index("c")
        s = lax.axis_index("s")
        wid = s * NC + c

        r0 = s * ROWS_T
        pltpu.sync_copy(zeros_hbm.at[pl.ds(r0, ROWS_T)], acc.at[pl.ds(r0, ROWS_T)])

        @pl.when(s == NS - 1)
        def _():
            pltpu.sync_copy(zeros_hbm.at[pl.ds(16 * ROWS_T, ROWS_LAST - ROWS_T)],
                            acc.at[pl.ds(16 * ROWS_T, ROWS_LAST - ROWS_T)])

        plsc.subcore_barrier()

        eoff = wid * EPW

        def chunk(j, _):
            base = eoff + j * K
            pltpu.sync_copy(src_hbm.at[pl.ds(base, K)], src_v)
            pltpu.sync_copy(dst_hbm.at[pl.ds(base, K)], dst_v)
            pltpu.sync_copy(w_hbm.at[pl.ds(base, K)], w_v)
            pltpu.async_copy(h_hbm.at[src_v], rows_v, sem).wait()
            for g in range(K // 16):
                wv = w_v[pl.ds(g * 16, 16)]
                for e in range(16):
                    wb = _broadcast_lane(wv, e)
                    ge = g * 16 + e
                    for f in range(nfv):
                        sl = pl.ds(f * 16, 16)
                        rows_v[ge, sl] = rows_v[ge, sl] * wb
            pltpu.sync_copy(rows_v, acc.at[dst_v], add=True)
            return ()

        lax.fori_loop(0, NCHUNK, chunk, (), unroll=False)

        plsc.subcore_barrier()

        pltpu.sync_copy(acc.at[pl.ds(r0, ROWS_T)], out_hbm.at[c, pl.ds(r0, ROWS_T)])

        @pl.when(s == NS - 1)
        def _():
            pltpu.sync_copy(acc.at[pl.ds(16 * ROWS_T, ROWS_LAST - ROWS_T)],
                            out_hbm.at[c, pl.ds(16 * ROWS_T, ROWS_LAST - ROWS_T)])

    mesh = plsc.VectorSubcoreMesh(core_axis_name="c", subcore_axis_name="s")
    return pl.kernel(
        body,
        out_type=jax.ShapeDtypeStruct((NC, N, feat), jnp.float32),
        mesh=mesh,
        compiler_params=pltpu.CompilerParams(use_tc_tiling_on_sc=(feat == 128)),
        scratch_types=[
            pltpu.VMEM_SHARED((N, feat), jnp.float32),
            pltpu.VMEM((K,), jnp.int32),
            pltpu.VMEM((K,), jnp.int32),
            pltpu.VMEM((K,), jnp.float32),
            pltpu.VMEM((K, feat), jnp.float32),
            pltpu.SemaphoreType.DMA,
        ],
    )


_sc_agg_128 = _make_sc_aggregate(LATENT)
_sc_agg_16 = _make_sc_aggregate(CLASSES)


def _mm_body(x_ref, w_ref, o_ref):
    o_ref[...] = jnp.dot(x_ref[...], w_ref[...],
                         preferred_element_type=jnp.float32)


def _matmul(x, w):
    n, k = x.shape
    m = w.shape[1]
    blk = 400
    return pl.pallas_call(
        _mm_body,
        grid=(n // blk,),
        in_specs=[pl.BlockSpec((blk, k), lambda i: (i, 0)),
                  pl.BlockSpec((k, m), lambda i: (0, 0))],
        out_specs=pl.BlockSpec((blk, m), lambda i: (i, 0)),
        out_shape=jax.ShapeDtypeStruct((n, m), jnp.float32),
    )(x, w)


def _kl(o, half):
    mean = o[:, :half]
    std = jax.nn.softplus(o[:, half:]) + 1e-10
    return -jnp.log(std) + (std * std + mean * mean) / 2.0 - 0.5


def _combine1_body(p0_ref, p1_ref, b_ref, w2_ref, out1_ref, ixz_ref, h2_ref):
    o = p0_ref[...] + p1_ref[...] + b_ref[...]
    out1_ref[...] = o
    ixz_ref[...] = _kl(o, LATENT // 2)
    h2_ref[...] = jnp.dot(o, w2_ref[...], preferred_element_type=jnp.float32)


def _combine1(p0, p1, b1, W2):
    blk = 400
    return pl.pallas_call(
        _combine1_body,
        grid=(N // blk,),
        in_specs=[pl.BlockSpec((blk, LATENT), lambda i: (i, 0)),
                  pl.BlockSpec((blk, LATENT), lambda i: (i, 0)),
                  pl.BlockSpec((1, LATENT), lambda i: (0, 0)),
                  pl.BlockSpec((LATENT, CLASSES), lambda i: (0, 0))],
        out_specs=[pl.BlockSpec((blk, LATENT), lambda i: (i, 0)),
                   pl.BlockSpec((blk, LATENT // 2), lambda i: (i, 0)),
                   pl.BlockSpec((blk, CLASSES), lambda i: (i, 0))],
        out_shape=[jax.ShapeDtypeStruct((N, LATENT), jnp.float32),
                   jax.ShapeDtypeStruct((N, LATENT // 2), jnp.float32),
                   jax.ShapeDtypeStruct((N, CLASSES), jnp.float32)],
    )(p0, p1, b1.reshape(1, LATENT), W2)


def _combine2_body(p0_ref, p1_ref, b_ref, out2_ref, ixz_ref):
    o = p0_ref[...] + p1_ref[...] + b_ref[...]
    out2_ref[...] = o
    ixz_ref[...] = _kl(o, CLASSES // 2)


def _combine2(p0, p1, b2):
    blk = 1000
    return pl.pallas_call(
        _combine2_body,
        grid=(N // blk,),
        in_specs=[pl.BlockSpec((blk, CLASSES), lambda i: (i, 0)),
                  pl.BlockSpec((blk, CLASSES), lambda i: (i, 0)),
                  pl.BlockSpec((1, CLASSES), lambda i: (0, 0))],
        out_specs=[pl.BlockSpec((blk, CLASSES), lambda i: (i, 0)),
                   pl.BlockSpec((blk, CLASSES // 2), lambda i: (i, 0))],
        out_shape=[jax.ShapeDtypeStruct((N, CLASSES), jnp.float32),
                   jax.ShapeDtypeStruct((N, CLASSES // 2), jnp.float32)],
    )(p0, p1, b2.reshape(1, CLASSES))


def kernel(x, edge_index, edge_attr, W1, b1, W2, b2):
    src = edge_index[0].astype(jnp.int32)
    dst = edge_index[1].astype(jnp.int32)
    w = edge_attr.astype(jnp.float32)
    zeros128 = jnp.zeros((N, LATENT), jnp.float32)
    zeros16 = jnp.zeros((N, CLASSES), jnp.float32)

    h1 = _matmul(x, W1)
    p1 = _sc_agg_128(h1, src, dst, w, zeros128)
    out1, ixz1, h2 = _combine1(p1[0], p1[1], b1, W2)
    p2 = _sc_agg_16(h2, src, dst, w, zeros16)
    out2, ixz2 = _combine2(p2[0], p2[1], b2)

    skl1 = jnp.zeros_like(ixz1)
    skl2 = jnp.zeros_like(ixz2)
    return (out2, out1, ixz1, skl1, ixz2, skl2)
